# baseline (speedup 1.0000x reference)
import functools

import jax
import jax.numpy as jnp
from jax import lax
from jax.experimental import pallas as pl
from jax.experimental.pallas import tpu as pltpu
from jax.experimental.pallas import tpu_sc as plsc

N = 50000
E = 800000
DEG = 16
H = 32
F = 8
DEPTH = 3
NMOL = 512
LMAX = 200

NW = 32
N_PAD = 51200
N4 = N_PAD // 4
N4_BLK = 1280
C_W = N_PAD // NW
CHUNK = 64
N_CHUNKS = C_W // CHUNK
IDXROW = 128
E_TAB = E + 32
E_W = E // NW
E_CHUNK = 1000
E_CHUNKS = E_W // E_CHUNK

MOL_W = NMOL // NW
MOL_IDX_ROWS = MOL_W * LMAX // IDXROW
MOL_IDX_STRIDE = 32


@functools.lru_cache(maxsize=None)
def _build_sc_edge():
    @functools.partial(
        pl.kernel,
        out_type=jax.ShapeDtypeStruct((E_TAB, H), jnp.float32),
        mesh=_sc_mesh(),
        compiler_params=pltpu.CompilerParams(use_tc_tiling_on_sc=False),
        scratch_types=[
            pltpu.VMEM((16, H), jnp.float32),
            pltpu.VMEM((2, E_CHUNK // 2 + 4, 2 * F), jnp.float32),
            pltpu.VMEM((2, E_CHUNK + 8), jnp.float32),
            pltpu.VMEM((2, E_CHUNK + 8, H), jnp.float32),
            pltpu.SemaphoreType.DMA,
            pltpu.SemaphoreType.DMA,
            pltpu.SemaphoreType.DMA,
            pltpu.SemaphoreType.DMA,
        ],
    )
    def _sc_edge(fdg2_hbm, rij_hbm, wib_hbm, rtab_hbm, w_v, fdg_v, rij_v, r_v,
                 sem_i0, sem_i1, sem_o0, sem_o1):
        wid = _worker_id()
        e_base = wid * E_W
        sem_i = (sem_i0, sem_i1)
        sem_o = (sem_o0, sem_o1)
        pltpu.sync_copy(wib_hbm, w_v)
        wv = [(w_v[k, pl.ds(0, 16)], w_v[k, pl.ds(16, 16)])
              for k in range(F + 1)]

        def in_descs(b, ci):
            e0 = pl.multiple_of(e_base + ci * E_CHUNK, E_CHUNK)
            return [
                pltpu.make_async_copy(
                    fdg2_hbm.at[pl.ds(e0 // 2, E_CHUNK // 2)],
                    fdg_v.at[b, pl.ds(0, E_CHUNK // 2)], sem_i[b]),
                pltpu.make_async_copy(
                    rij_hbm.at[pl.ds(e0, E_CHUNK)],
                    rij_v.at[b, pl.ds(0, E_CHUNK)], sem_i[b]),
            ]

        def out_desc(b, ci):
            e0 = pl.multiple_of(e_base + ci * E_CHUNK, E_CHUNK)
            return pltpu.make_async_copy(
                r_v.at[b, pl.ds(0, E_CHUNK)],
                rtab_hbm.at[pl.ds(e0, E_CHUNK)], sem_o[b])

        def fire(b, ci):
            for d in in_descs(b, ci):
                d.start()

        def compute(b, ci, wait_prev_out):
            for d in in_descs(b, ci):
                d.wait()
            if wait_prev_out:
                out_desc(b, ci - 2).wait()

            @pl.loop(0, (E_CHUNK + 8) // 16)
            def _(g):
                rv16 = rij_v[b, pl.ds(g * 16, 16)]
                for p8 in range(8):
                    pair = g * 8 + p8
                    v = fdg_v[b, pair, pl.ds(0, 16)]
                    for half in range(2):
                        row = pair * 2 + half
                        c0 = rv16[2 * p8 + half] * wv[F][0]
                        c1 = rv16[2 * p8 + half] * wv[F][1]
                        for k in range(F):
                            xk = v[half * F + k]
                            c0 = c0 + xk * wv[k][0]
                            c1 = c1 + xk * wv[k][1]
                        r_v[b, row, pl.ds(0, 16)] = jnp.maximum(c0, 0.0)
                        r_v[b, row, pl.ds(16, 16)] = jnp.maximum(c1, 0.0)

            out_desc(b, ci).start()

        fire(0, 0)
        fire(1, 1)
        compute(0, 0, False)
        fire(0, 2)
        compute(1, 1, False)

        @pl.loop(1, E_CHUNKS // 2)
        def _(t):
            fire(1, 2 * t + 1)
            compute(0, 2 * t, True)
            fire(0, 2 * t + 2)
            compute(1, 2 * t + 1, True)

        compute(0, E_CHUNKS - 1, True)
        out_desc(1, E_CHUNKS - 2).wait()
        out_desc(0, E_CHUNKS - 1).wait()

        @pl.when(wid == 0)
        def _():
            @pl.loop(0, 32)
            def _(j):
                r_v[0, j, pl.ds(0, 16)] = jnp.zeros((16,), jnp.float32)
                r_v[0, j, pl.ds(16, 16)] = jnp.zeros((16,), jnp.float32)
            pltpu.sync_copy(r_v.at[0, pl.ds(0, 32)],
                            rtab_hbm.at[pl.ds(E, 32)])

    return _sc_edge


def _gru_math(i, h, agg_ref, wh_ref, wz_ref, wr_ref, wc_ref, bz_ref, br_ref,
              bc_ref, out_ref):
    msg = jnp.dot(agg_ref[...], wh_ref[...], preferred_element_type=jnp.float32)
    z = jax.nn.sigmoid(
        jnp.dot(msg, wz_ref[0], preferred_element_type=jnp.float32)
        + jnp.dot(h, wz_ref[1], preferred_element_type=jnp.float32)
        + bz_ref[...])
    r = jax.nn.sigmoid(
        jnp.dot(msg, wr_ref[0], preferred_element_type=jnp.float32)
        + jnp.dot(h, wr_ref[1], preferred_element_type=jnp.float32)
        + br_ref[...])
    cand = jnp.tanh(
        jnp.dot(msg, wc_ref[0], preferred_element_type=jnp.float32)
        + jnp.dot(r * h, wc_ref[1], preferred_element_type=jnp.float32)
        + bc_ref[...])
    hn = (1.0 - z) * h + z * cand
    row = i * N4_BLK + lax.broadcasted_iota(jnp.int32, (N4_BLK, 1), 0)
    lane = lax.broadcasted_iota(jnp.int32, (N4_BLK, 4 * H), 1)
    node = row * 4 + lane // H
    out_ref[...] = jnp.where(node < N, hn, 0.0)


def _gru0_body(tf_ref, agg_ref, wia_ref, wh_ref, wz_ref, wr_ref, wc_ref,
               bz_ref, br_ref, bc_ref, out_ref):
    i = pl.program_id(0)
    h = jnp.dot(tf_ref[...], wia_ref[...], preferred_element_type=jnp.float32)
    _gru_math(i, h, agg_ref, wh_ref, wz_ref, wr_ref, wc_ref, bz_ref, br_ref,
              bc_ref, out_ref)


def _grun_body(h_ref, agg_ref, wh_ref, wz_ref, wr_ref, wc_ref, bz_ref,
               br_ref, bc_ref, out_ref):
    i = pl.program_id(0)
    _gru_math(i, h_ref[...], agg_ref, wh_ref, wz_ref, wr_ref, wc_ref, bz_ref,
              br_ref, bc_ref, out_ref)


def _node_spec():
    return pl.BlockSpec((N4_BLK, 4 * H), lambda i: (i, 0))


def _full(shape):
    return pl.BlockSpec(shape, lambda i: tuple(0 for _ in shape))


def _gru_first(tf4, agg4, wia_bd, wh_bd, wz, wr, wc, bz, br, bc):
    return pl.pallas_call(
        _gru0_body,
        grid=(N4 // N4_BLK,),
        in_specs=[
            pl.BlockSpec((N4_BLK, 4 * F), lambda i: (i, 0)),
            _node_spec(),
            _full((4 * F, 4 * H)),
            _full((4 * H, 4 * H)),
            _full((2, 4 * H, 4 * H)),
            _full((2, 4 * H, 4 * H)),
            _full((2, 4 * H, 4 * H)),
            _full((1, 4 * H)),
            _full((1, 4 * H)),
            _full((1, 4 * H)),
        ],
        out_specs=_node_spec(),
        out_shape=jax.ShapeDtypeStruct((N4, 4 * H), jnp.float32),
    )(tf4, agg4, wia_bd, wh_bd, wz, wr, wc, bz, br, bc)


def _gru_next(h4, agg4, wh_bd, wz, wr, wc, bz, br, bc):
    return pl.pallas_call(
        _grun_body,
        grid=(N4 // N4_BLK,),
        in_specs=[
            _node_spec(),
            _node_spec(),
            _full((4 * H, 4 * H)),
            _full((2, 4 * H, 4 * H)),
            _full((2, 4 * H, 4 * H)),
            _full((2, 4 * H, 4 * H)),
            _full((1, 4 * H)),
            _full((1, 4 * H)),
            _full((1, 4 * H)),
        ],
        out_specs=_node_spec(),
        out_shape=jax.ShapeDtypeStruct((N4, 4 * H), jnp.float32),
    )(h4, agg4, wh_bd, wz, wr, wc, bz, br, bc)


def _sc_mesh():
    return plsc.VectorSubcoreMesh(core_axis_name="c", subcore_axis_name="s")


def _worker_id():
    return lax.axis_index("s") * 2 + lax.axis_index("c")


def _reduce_groups(rows_v, acc_v, b, count, group):
    @pl.loop(0, count)
    def _(j):
        base = j * group
        a0 = rows_v[b, base, pl.ds(0, 16)]
        a1 = rows_v[b, base, pl.ds(16, 16)]
        for d in range(1, group):
            a0 = a0 + rows_v[b, base + d, pl.ds(0, 16)]
            a1 = a1 + rows_v[b, base + d, pl.ds(16, 16)]
        acc_v[b, j, pl.ds(0, 16)] = a0
        acc_v[b, j, pl.ds(16, 16)] = a1


@functools.lru_cache(maxsize=None)
def _build_sc_agg_first():
    @functools.partial(
        pl.kernel,
        out_type=(jax.ShapeDtypeStruct((N_PAD, H), jnp.float32),
                  jax.ShapeDtypeStruct((N_PAD * DEG // IDXROW, IDXROW),
                                       jnp.int32)),
        mesh=_sc_mesh(),
        compiler_params=pltpu.CompilerParams(use_tc_tiling_on_sc=False),
        scratch_types=[
            pltpu.VMEM((2, CHUNK * DEG // IDXROW, IDXROW), jnp.int32),
            pltpu.VMEM((2, CHUNK * DEG, H), jnp.float32),
            pltpu.VMEM((2, CHUNK * DEG // IDXROW, IDXROW), jnp.int32),
            pltpu.VMEM((2, CHUNK, H), jnp.float32),
            pltpu.SemaphoreType.DMA,
            pltpu.SemaphoreType.DMA,
            pltpu.SemaphoreType.DMA,
            pltpu.SemaphoreType.DMA,
        ],
    )
    def _sc_agg_first(rtab_hbm, bsidx_hbm, sutab_hbm, agg_hbm, comp_hbm,
                      idx_v, rows_v, comp_v, acc_v, sem_r0, sem_r1, sem_c0,
                      sem_c1):
        wid = _worker_id()
        node0 = wid * C_W
        sem_r = (sem_r0, sem_r1)
        sem_c = (sem_c0, sem_c1)
        nrows = CHUNK * DEG // IDXROW

        def bounds(ci):
            nb = pl.multiple_of(node0 + ci * CHUNK, CHUNK)
            return nb, pl.multiple_of(nb * DEG // IDXROW, 8)

        def gather_descs(b):
            descs = []
            for j in range(nrows):
                descs.append(pltpu.make_async_copy(
                    rtab_hbm.at[idx_v.at[b, j]],
                    rows_v.at[b, pl.ds(j * IDXROW, IDXROW)], sem_r[b]))
                descs.append(pltpu.make_async_copy(
                    sutab_hbm.at[idx_v.at[b, j]],
                    comp_v.at[b, j], sem_c[b]))
            return descs

        def fire(b, ci):
            _, irow = bounds(ci)
            pltpu.sync_copy(bsidx_hbm.at[pl.ds(irow, nrows)], idx_v.at[b])
            for g in gather_descs(b):
                g.start()

        def finish(b, ci):
            nb, irow = bounds(ci)
            for g in gather_descs(b):
                g.wait()
            _reduce_groups(rows_v, acc_v, b, CHUNK, DEG)
            pltpu.sync_copy(acc_v.at[b], agg_hbm.at[pl.ds(nb, CHUNK)])
            pltpu.sync_copy(comp_v.at[b], comp_hbm.at[pl.ds(irow, nrows)])

        fire(0, 0)

        @pl.loop(0, N_CHUNKS // 2)
        def _(t):
            fire(1, 2 * t + 1)
            finish(0, 2 * t)
            fire(0, 2 * t + 2)
            finish(1, 2 * t + 1)

        finish(0, N_CHUNKS - 1)

    return _sc_agg_first


@functools.lru_cache(maxsize=None)
def _build_sc_agg():
    @functools.partial(
        pl.kernel,
        out_type=jax.ShapeDtypeStruct((N_PAD, H), jnp.float32),
        mesh=_sc_mesh(),
        compiler_params=pltpu.CompilerParams(use_tc_tiling_on_sc=False),
        scratch_types=[
            pltpu.VMEM((2, CHUNK * DEG // IDXROW, IDXROW), jnp.int32),
            pltpu.VMEM((2, CHUNK * DEG, H), jnp.float32),
            pltpu.VMEM((2, CHUNK, H), jnp.float32),
            pltpu.SemaphoreType.DMA,
            pltpu.SemaphoreType.DMA,
        ],
    )
    def _sc_agg(htab_hbm, comp_hbm, agg_hbm, idx_v, rows_v, acc_v, sem_r0,
                sem_r1):
        wid = _worker_id()
        node0 = wid * C_W
        sem_r = (sem_r0, sem_r1)
        nrows = CHUNK * DEG // IDXROW

        def bounds(ci):
            nb = pl.multiple_of(node0 + ci * CHUNK, CHUNK)
            return nb, pl.multiple_of(nb * DEG // IDXROW, 8)

        def gather_descs(b):
            return [pltpu.make_async_copy(
                        htab_hbm.at[idx_v.at[b, j]],
                        rows_v.at[b, pl.ds(j * IDXROW, IDXROW)], sem_r[b])
                    for j in range(nrows)]

        def fire(b, ci):
            _, irow = bounds(ci)
            pltpu.sync_copy(comp_hbm.at[pl.ds(irow, nrows)], idx_v.at[b])
            for g in gather_descs(b):
                g.start()

        def finish(b, ci):
            nb, _ = bounds(ci)
            for g in gather_descs(b):
                g.wait()
            _reduce_groups(rows_v, acc_v, b, CHUNK, DEG)
            pltpu.sync_copy(acc_v.at[b], agg_hbm.at[pl.ds(nb, CHUNK)])

        fire(0, 0)

        @pl.loop(0, N_CHUNKS // 2)
        def _(t):
            fire(1, 2 * t + 1)
            finish(0, 2 * t)
            fire(0, 2 * t + 2)
            finish(1, 2 * t + 1)

        finish(0, N_CHUNKS - 1)

    return _sc_agg


@functools.lru_cache(maxsize=None)
def _build_sc_mol():
    @functools.partial(
        pl.kernel,
        out_type=jax.ShapeDtypeStruct((NMOL, H), jnp.float32),
        mesh=_sc_mesh(),
        compiler_params=pltpu.CompilerParams(use_tc_tiling_on_sc=False),
        scratch_types=[
            pltpu.VMEM((MOL_IDX_STRIDE, IDXROW), jnp.int32),
            pltpu.VMEM((MOL_W * LMAX, H), jnp.float32),
            pltpu.VMEM((MOL_W, H), jnp.float32),
            pltpu.SemaphoreType.DMA,
        ],
    )
    def _sc_mol(htab_hbm, lidx_hbm, out_hbm, idx_v, rows_v, acc_v, sem_r):
        wid = _worker_id()
        irow = pl.multiple_of(wid * MOL_IDX_STRIDE, 8)
        pltpu.sync_copy(lidx_hbm.at[pl.ds(irow, MOL_IDX_STRIDE)], idx_v)

        @pl.loop(0, MOL_IDX_ROWS // 5)
        def _(t):
            gathers = []
            for j in range(5):
                r = t * 5 + j
                gathers.append(pltpu.async_copy(
                    htab_hbm.at[idx_v.at[r]],
                    rows_v.at[pl.ds(r * IDXROW, IDXROW)], sem_r))
            for g in gathers:
                g.wait()

        @pl.loop(0, MOL_W)
        def _(j):
            acc_v[j, pl.ds(0, 16)] = jnp.zeros((16,), jnp.float32)
            acc_v[j, pl.ds(16, 16)] = jnp.zeros((16,), jnp.float32)

            @pl.loop(0, LMAX // 8)
            def _(t):
                base = j * LMAX + t * 8
                a0 = rows_v[base, pl.ds(0, 16)]
                a1 = rows_v[base, pl.ds(16, 16)]
                for d in range(1, 8):
                    a0 = a0 + rows_v[base + d, pl.ds(0, 16)]
                    a1 = a1 + rows_v[base + d, pl.ds(16, 16)]
                acc_v[j, pl.ds(0, 16)] = acc_v[j, pl.ds(0, 16)] + a0
                acc_v[j, pl.ds(16, 16)] = acc_v[j, pl.ds(16, 16)] + a1

        pltpu.sync_copy(
            acc_v, out_hbm.at[pl.ds(pl.multiple_of(wid * MOL_W, 8), MOL_W)])

    return _sc_mol


def _bd4(w):
    return jnp.kron(jnp.eye(4, dtype=w.dtype), w)


def kernel(target_features, feature_dist_graph, rij_dist_pairs, b_scope,
           start_end_env, l_scope, scope_update, scope_update_lig,
           W_i_a, W_i_b, W_h, gru_Wz, gru_Wr, gru_Wh, gru_bz, gru_br, gru_bh):
    bs = jnp.concatenate(
        [b_scope, jnp.zeros((N_PAD - N, DEG), b_scope.dtype)], axis=0)
    bs_idx = jnp.where(bs > 0, bs - 1, E).astype(jnp.int32).reshape(-1, IDXROW)
    su_tab = jnp.concatenate(
        [scope_update.astype(jnp.int32),
         jnp.full((8,), N, jnp.int32)], axis=0)
    l_idx = jnp.where(l_scope > 0, l_scope - 1, N).astype(jnp.int32)
    l_idx = l_idx.reshape(NW, MOL_IDX_ROWS, IDXROW)
    l_idx = jnp.concatenate(
        [l_idx,
         jnp.full((NW, MOL_IDX_STRIDE - MOL_IDX_ROWS, IDXROW), N, jnp.int32)],
        axis=1).reshape(NW * MOL_IDX_STRIDE, IDXROW)

    tf4 = jnp.concatenate(
        [target_features,
         jnp.zeros((N_PAD - N, F), target_features.dtype)], axis=0)
    tf4 = tf4.reshape(N4, 4 * F)
    wib_pad = jnp.concatenate(
        [W_i_b, jnp.zeros((16 - (F + 1), H), W_i_b.dtype)], axis=0)
    wia_bd = _bd4(W_i_a)
    wh_bd = _bd4(W_h)
    wz = jnp.stack([jax.vmap(_bd4)(gru_Wz[:, :H]),
                    jax.vmap(_bd4)(gru_Wz[:, H:])], axis=1)
    wr = jnp.stack([jax.vmap(_bd4)(gru_Wr[:, :H]),
                    jax.vmap(_bd4)(gru_Wr[:, H:])], axis=1)
    wc = jnp.stack([jax.vmap(_bd4)(gru_Wh[:, :H]),
                    jax.vmap(_bd4)(gru_Wh[:, H:])], axis=1)
    bz = jnp.tile(gru_bz, (1, 4))[:, None, :]
    br = jnp.tile(gru_br, (1, 4))[:, None, :]
    bc = jnp.tile(gru_bh, (1, 4))[:, None, :]

    fdg2 = feature_dist_graph.reshape(E // 2, 2 * F)
    r_tab = _build_sc_edge()(fdg2, rij_dist_pairs, wib_pad)
    agg, comp = _build_sc_agg_first()(r_tab, bs_idx, su_tab)
    h4 = _gru_first(tf4, agg.reshape(N4, 4 * H), wia_bd, wh_bd,
                    wz[0], wr[0], wc[0], bz[0], br[0], bc[0])
    for i in range(1, DEPTH):
        agg = _build_sc_agg()(h4.reshape(N_PAD, H), comp)
        h4 = _gru_next(h4, agg.reshape(N4, 4 * H), wh_bd,
                       wz[i], wr[i], wc[i], bz[i], br[i], bc[i])
    return _build_sc_mol()(h4.reshape(N_PAD, H), l_idx)


# speedup vs baseline: 1.0989x; 1.0989x over previous
import functools

import jax
import jax.numpy as jnp
from jax import lax
from jax.experimental import pallas as pl
from jax.experimental.pallas import tpu as pltpu
from jax.experimental.pallas import tpu_sc as plsc

N = 50000
E = 800000
DEG = 16
H = 32
F = 8
DEPTH = 3
NMOL = 512
LMAX = 200

NW = 32
N_PAD = 51200
N4 = N_PAD // 4
N4_BLK = 1280
C_W = N_PAD // NW
CHUNK = 64
N_CHUNKS = C_W // CHUNK
IDXROW = 128
E4 = E // 4
E4_BLK = 2000
E4_PAD = 202000
E_TAB = E4_PAD * 4

MOL_W = NMOL // NW
MOL_IDX_ROWS = MOL_W * LMAX // IDXROW
MOL_IDX_STRIDE = 32


def _edge_emb_body(fdg_ref, rij_ref, w8_ref, w9_ref, out_ref):
    i = pl.program_id(0)
    v = jnp.dot(fdg_ref[...], w8_ref[...], preferred_element_type=jnp.float32)
    v = v + jnp.dot(rij_ref[...], w9_ref[...],
                    preferred_element_type=jnp.float32)
    v = jnp.maximum(v, 0.0)
    row = i * E4_BLK + lax.broadcasted_iota(jnp.int32, (E4_BLK, 1), 0)
    out_ref[...] = jnp.where(row < E4, v, 0.0)


def _edge_emb(fdg4, rij4, w8bd, w9bd):
    nblk = E4_PAD // E4_BLK
    last = E4 // E4_BLK - 1
    return pl.pallas_call(
        _edge_emb_body,
        grid=(nblk,),
        in_specs=[
            pl.BlockSpec((E4_BLK, 4 * F), lambda i: (jnp.minimum(i, last), 0)),
            pl.BlockSpec((E4_BLK, 4), lambda i: (jnp.minimum(i, last), 0)),
            pl.BlockSpec((4 * F, 4 * H), lambda i: (0, 0)),
            pl.BlockSpec((4, 4 * H), lambda i: (0, 0)),
        ],
        out_specs=pl.BlockSpec((E4_BLK, 4 * H), lambda i: (i, 0)),
        out_shape=jax.ShapeDtypeStruct((E4_PAD, 4 * H), jnp.float32),
    )(fdg4, rij4, w8bd, w9bd)


def _gru_math(i, h, agg_ref, wh_ref, wz_ref, wr_ref, wc_ref, bz_ref, br_ref,
              bc_ref, out_ref):
    msg = jnp.dot(agg_ref[...], wh_ref[...], preferred_element_type=jnp.float32)
    z = jax.nn.sigmoid(
        jnp.dot(msg, wz_ref[0], preferred_element_type=jnp.float32)
        + jnp.dot(h, wz_ref[1], preferred_element_type=jnp.float32)
        + bz_ref[...])
    r = jax.nn.sigmoid(
        jnp.dot(msg, wr_ref[0], preferred_element_type=jnp.float32)
        + jnp.dot(h, wr_ref[1], preferred_element_type=jnp.float32)
        + br_ref[...])
    cand = jnp.tanh(
        jnp.dot(msg, wc_ref[0], preferred_element_type=jnp.float32)
        + jnp.dot(r * h, wc_ref[1], preferred_element_type=jnp.float32)
        + bc_ref[...])
    hn = (1.0 - z) * h + z * cand
    row = i * N4_BLK + lax.broadcasted_iota(jnp.int32, (N4_BLK, 1), 0)
    lane = lax.broadcasted_iota(jnp.int32, (N4_BLK, 4 * H), 1)
    node = row * 4 + lane // H
    out_ref[...] = jnp.where(node < N, hn, 0.0)


def _gru0_body(tf_ref, agg_ref, wia_ref, wh_ref, wz_ref, wr_ref, wc_ref,
               bz_ref, br_ref, bc_ref, out_ref):
    i = pl.program_id(0)
    h = jnp.dot(tf_ref[...], wia_ref[...], preferred_element_type=jnp.float32)
    _gru_math(i, h, agg_ref, wh_ref, wz_ref, wr_ref, wc_ref, bz_ref, br_ref,
              bc_ref, out_ref)


def _grun_body(h_ref, agg_ref, wh_ref, wz_ref, wr_ref, wc_ref, bz_ref,
               br_ref, bc_ref, out_ref):
    i = pl.program_id(0)
    _gru_math(i, h_ref[...], agg_ref, wh_ref, wz_ref, wr_ref, wc_ref, bz_ref,
              br_ref, bc_ref, out_ref)


def _node_spec():
    return pl.BlockSpec((N4_BLK, 4 * H), lambda i: (i, 0))


def _full(shape):
    return pl.BlockSpec(shape, lambda i: tuple(0 for _ in shape))


def _gru_first(tf4, agg4, wia_bd, wh_bd, wz, wr, wc, bz, br, bc):
    return pl.pallas_call(
        _gru0_body,
        grid=(N4 // N4_BLK,),
        in_specs=[
            pl.BlockSpec((N4_BLK, 4 * F), lambda i: (i, 0)),
            _node_spec(),
            _full((4 * F, 4 * H)),
            _full((4 * H, 4 * H)),
            _full((2, 4 * H, 4 * H)),
            _full((2, 4 * H, 4 * H)),
            _full((2, 4 * H, 4 * H)),
            _full((1, 4 * H)),
            _full((1, 4 * H)),
            _full((1, 4 * H)),
        ],
        out_specs=_node_spec(),
        out_shape=jax.ShapeDtypeStruct((N4, 4 * H), jnp.float32),
    )(tf4, agg4, wia_bd, wh_bd, wz, wr, wc, bz, br, bc)


def _gru_next(h4, agg4, wh_bd, wz, wr, wc, bz, br, bc):
    return pl.pallas_call(
        _grun_body,
        grid=(N4 // N4_BLK,),
        in_specs=[
            _node_spec(),
            _node_spec(),
            _full((4 * H, 4 * H)),
            _full((2, 4 * H, 4 * H)),
            _full((2, 4 * H, 4 * H)),
            _full((2, 4 * H, 4 * H)),
            _full((1, 4 * H)),
            _full((1, 4 * H)),
            _full((1, 4 * H)),
        ],
        out_specs=_node_spec(),
        out_shape=jax.ShapeDtypeStruct((N4, 4 * H), jnp.float32),
    )(h4, agg4, wh_bd, wz, wr, wc, bz, br, bc)


def _sc_mesh():
    return plsc.VectorSubcoreMesh(core_axis_name="c", subcore_axis_name="s")


def _worker_id():
    return lax.axis_index("s") * 2 + lax.axis_index("c")


def _reduce_groups(rows_v, acc_v, b, count, group):
    @pl.loop(0, count)
    def _(j):
        base = j * group
        a0 = rows_v[b, base, pl.ds(0, 16)]
        a1 = rows_v[b, base, pl.ds(16, 16)]
        for d in range(1, group):
            a0 = a0 + rows_v[b, base + d, pl.ds(0, 16)]
            a1 = a1 + rows_v[b, base + d, pl.ds(16, 16)]
        acc_v[b, j, pl.ds(0, 16)] = a0
        acc_v[b, j, pl.ds(16, 16)] = a1


@functools.lru_cache(maxsize=None)
def _build_sc_agg_first():
    @functools.partial(
        pl.kernel,
        out_type=(jax.ShapeDtypeStruct((N_PAD, H), jnp.float32),
                  jax.ShapeDtypeStruct((N_PAD * DEG,), jnp.int32)),
        mesh=_sc_mesh(),
        compiler_params=pltpu.CompilerParams(use_tc_tiling_on_sc=False),
        scratch_types=[
            pltpu.VMEM((2, CHUNK * DEG), jnp.int32),
            pltpu.VMEM((2, CHUNK * DEG, H), jnp.float32),
            pltpu.VMEM((2, CHUNK * DEG), jnp.int32),
            pltpu.VMEM((2, CHUNK, H), jnp.float32),
            pltpu.SemaphoreType.DMA,
            pltpu.SemaphoreType.DMA,
            pltpu.SemaphoreType.DMA,
            pltpu.SemaphoreType.DMA,
        ],
    )
    def _sc_agg_first(rtab_hbm, bsidx_hbm, sutab_hbm, agg_hbm, comp_hbm,
                      idx_v, rows_v, comp_v, acc_v, sem_r0, sem_r1, sem_c0,
                      sem_c1):
        wid = _worker_id()
        node0 = wid * C_W
        sem_r = (sem_r0, sem_r1)
        sem_c = (sem_c0, sem_c1)
        nrows = CHUNK * DEG // IDXROW

        def bounds(ci):
            nb = pl.multiple_of(node0 + ci * CHUNK, CHUNK)
            return nb, pl.multiple_of(nb * DEG, CHUNK * DEG)

        def gather_descs(b):
            descs = []
            for j in range(nrows):
                descs.append(pltpu.make_async_copy(
                    rtab_hbm.at[idx_v.at[b, pl.ds(j * IDXROW, IDXROW)]],
                    rows_v.at[b, pl.ds(j * IDXROW, IDXROW)], sem_r[b]))
                descs.append(pltpu.make_async_copy(
                    sutab_hbm.at[idx_v.at[b, pl.ds(j * IDXROW, IDXROW)]],
                    comp_v.at[b, pl.ds(j * IDXROW, IDXROW)], sem_c[b]))
            return descs

        def fire(b, ci):
            _, ib = bounds(ci)
            pltpu.sync_copy(bsidx_hbm.at[pl.ds(ib, CHUNK * DEG)], idx_v.at[b])
            for g in gather_descs(b):
                g.start()

        def finish(b, ci):
            nb, ib = bounds(ci)
            for g in gather_descs(b):
                g.wait()
            _reduce_groups(rows_v, acc_v, b, CHUNK, DEG)
            pltpu.sync_copy(acc_v.at[b], agg_hbm.at[pl.ds(nb, CHUNK)])
            pltpu.sync_copy(comp_v.at[b], comp_hbm.at[pl.ds(ib, CHUNK * DEG)])

        fire(0, 0)

        @pl.loop(0, N_CHUNKS // 2)
        def _(t):
            fire(1, 2 * t + 1)
            finish(0, 2 * t)
            fire(0, 2 * t + 2)
            finish(1, 2 * t + 1)

        finish(0, N_CHUNKS - 1)

    return _sc_agg_first


@functools.lru_cache(maxsize=None)
def _build_sc_agg():
    @functools.partial(
        pl.kernel,
        out_type=jax.ShapeDtypeStruct((N_PAD, H), jnp.float32),
        mesh=_sc_mesh(),
        compiler_params=pltpu.CompilerParams(use_tc_tiling_on_sc=False),
        scratch_types=[
            pltpu.VMEM((2, CHUNK * DEG), jnp.int32),
            pltpu.VMEM((2, CHUNK * DEG, H), jnp.float32),
            pltpu.VMEM((2, CHUNK, H), jnp.float32),
            pltpu.SemaphoreType.DMA,
            pltpu.SemaphoreType.DMA,
        ],
    )
    def _sc_agg(htab_hbm, comp_hbm, agg_hbm, idx_v, rows_v, acc_v,
                sem_r0, sem_r1):
        wid = _worker_id()
        node0 = wid * C_W
        sem_r = (sem_r0, sem_r1)
        nrows = CHUNK * DEG // IDXROW

        def bounds(ci):
            nb = pl.multiple_of(node0 + ci * CHUNK, CHUNK)
            return nb, pl.multiple_of(nb * DEG, CHUNK * DEG)

        def gather_descs(b):
            return [pltpu.make_async_copy(
                        htab_hbm.at[idx_v.at[b, pl.ds(j * IDXROW, IDXROW)]],
                        rows_v.at[b, pl.ds(j * IDXROW, IDXROW)], sem_r[b])
                    for j in range(nrows)]

        def fire(b, ci):
            _, ib = bounds(ci)
            pltpu.sync_copy(comp_hbm.at[pl.ds(ib, CHUNK * DEG)], idx_v.at[b])
            for g in gather_descs(b):
                g.start()

        def finish(b, ci):
            nb, _ = bounds(ci)
            for g in gather_descs(b):
                g.wait()
            _reduce_groups(rows_v, acc_v, b, CHUNK, DEG)
            pltpu.sync_copy(acc_v.at[b], agg_hbm.at[pl.ds(nb, CHUNK)])

        fire(0, 0)

        @pl.loop(0, N_CHUNKS // 2)
        def _(t):
            fire(1, 2 * t + 1)
            finish(0, 2 * t)
            fire(0, 2 * t + 2)
            finish(1, 2 * t + 1)

        finish(0, N_CHUNKS - 1)

    return _sc_agg


@functools.lru_cache(maxsize=None)
def _build_sc_mol():
    @functools.partial(
        pl.kernel,
        out_type=jax.ShapeDtypeStruct((NMOL, H), jnp.float32),
        mesh=_sc_mesh(),
        compiler_params=pltpu.CompilerParams(use_tc_tiling_on_sc=False),
        scratch_types=[
            pltpu.VMEM((MOL_W * LMAX,), jnp.int32),
            pltpu.VMEM((MOL_W * LMAX, H), jnp.float32),
            pltpu.VMEM((MOL_W, H), jnp.float32),
            pltpu.SemaphoreType.DMA,
        ],
    )
    def _sc_mol(htab_hbm, lidx_hbm, out_hbm, idx_v, rows_v, acc_v, sem_r):
        wid = _worker_id()
        i0 = pl.multiple_of(wid * MOL_W * LMAX, 8)
        pltpu.sync_copy(lidx_hbm.at[pl.ds(i0, MOL_W * LMAX)], idx_v)

        @pl.loop(0, MOL_IDX_ROWS // 5)
        def _(t):
            gathers = []
            for j in range(5):
                r = t * 5 + j
                gathers.append(pltpu.async_copy(
                    htab_hbm.at[idx_v.at[pl.ds(r * IDXROW, IDXROW)]],
                    rows_v.at[pl.ds(r * IDXROW, IDXROW)], sem_r))
            for g in gathers:
                g.wait()

        @pl.loop(0, MOL_W)
        def _(j):
            acc_v[j, pl.ds(0, 16)] = jnp.zeros((16,), jnp.float32)
            acc_v[j, pl.ds(16, 16)] = jnp.zeros((16,), jnp.float32)

            @pl.loop(0, LMAX // 8)
            def _(t):
                base = j * LMAX + t * 8
                a0 = rows_v[base, pl.ds(0, 16)]
                a1 = rows_v[base, pl.ds(16, 16)]
                for d in range(1, 8):
                    a0 = a0 + rows_v[base + d, pl.ds(0, 16)]
                    a1 = a1 + rows_v[base + d, pl.ds(16, 16)]
                acc_v[j, pl.ds(0, 16)] = acc_v[j, pl.ds(0, 16)] + a0
                acc_v[j, pl.ds(16, 16)] = acc_v[j, pl.ds(16, 16)] + a1

        pltpu.sync_copy(
            acc_v, out_hbm.at[pl.ds(pl.multiple_of(wid * MOL_W, 8), MOL_W)])

    return _sc_mol


def _bd4(w):
    return jnp.kron(jnp.eye(4, dtype=w.dtype), w)


def kernel(target_features, feature_dist_graph, rij_dist_pairs, b_scope,
           start_end_env, l_scope, scope_update, scope_update_lig,
           W_i_a, W_i_b, W_h, gru_Wz, gru_Wr, gru_Wh, gru_bz, gru_br, gru_bh):
    bs = jnp.concatenate(
        [b_scope, jnp.zeros((N_PAD - N, DEG), b_scope.dtype)], axis=0)
    bs_idx = jnp.where(bs > 0, bs - 1, E).astype(jnp.int32).reshape(-1)
    su_tab = jnp.concatenate(
        [scope_update.astype(jnp.int32),
         jnp.full((8,), N, jnp.int32)], axis=0)
    l_idx = jnp.where(l_scope > 0, l_scope - 1, N).astype(jnp.int32).reshape(-1)

    tf4 = jnp.concatenate(
        [target_features,
         jnp.zeros((N_PAD - N, F), target_features.dtype)], axis=0)
    tf4 = tf4.reshape(N4, 4 * F)
    w8bd = _bd4(W_i_b[:F])
    w9bd = _bd4(W_i_b[F:F + 1])
    wia_bd = _bd4(W_i_a)
    wh_bd = _bd4(W_h)
    wz = jnp.stack([jax.vmap(_bd4)(gru_Wz[:, :H]),
                    jax.vmap(_bd4)(gru_Wz[:, H:])], axis=1)
    wr = jnp.stack([jax.vmap(_bd4)(gru_Wr[:, :H]),
                    jax.vmap(_bd4)(gru_Wr[:, H:])], axis=1)
    wc = jnp.stack([jax.vmap(_bd4)(gru_Wh[:, :H]),
                    jax.vmap(_bd4)(gru_Wh[:, H:])], axis=1)
    bz = jnp.tile(gru_bz, (1, 4))[:, None, :]
    br = jnp.tile(gru_br, (1, 4))[:, None, :]
    bc = jnp.tile(gru_bh, (1, 4))[:, None, :]

    fdg4 = feature_dist_graph.reshape(E4, 4 * F)
    rij4 = rij_dist_pairs.reshape(E4, 4)
    r_tab4 = _edge_emb(fdg4, rij4, w8bd, w9bd)
    r_tab = r_tab4.reshape(E_TAB, H)
    agg, comp = _build_sc_agg_first()(r_tab, bs_idx, su_tab)
    h4 = _gru_first(tf4, agg.reshape(N4, 4 * H), wia_bd, wh_bd,
                    wz[0], wr[0], wc[0], bz[0], br[0], bc[0])
    for i in range(1, DEPTH):
        agg = _build_sc_agg()(h4.reshape(N_PAD, H), comp)
        h4 = _gru_next(h4, agg.reshape(N4, 4 * H), wh_bd,
                       wz[i], wr[i], wc[i], bz[i], br[i], bc[i])
    return _build_sc_mol()(h4.reshape(N_PAD, H), l_idx)
